# baseline (speedup 1.0000x reference)
import functools

import jax
import jax.numpy as jnp
from jax import lax
from jax.experimental import pallas as pl
from jax.experimental.pallas import tpu as pltpu
from jax.experimental.pallas import tpu_sc as plsc

_B = 16384
_E = 64
_IDX_CHUNK = 128


def _make_gather():
    info = plsc.get_sparse_core_info()
    nc, ns = info.num_cores, info.num_subcores
    nw = nc * ns
    bpw = _B // nw
    nchunk = bpw // _IDX_CHUNK

    mesh = plsc.VectorSubcoreMesh(core_axis_name="c", subcore_axis_name="s")

    @functools.partial(
        pl.kernel,
        out_type=(
            jax.ShapeDtypeStruct((_B, _E), jnp.float32),
            jax.ShapeDtypeStruct((_B, _E), jnp.float32),
        ),
        mesh=mesh,
        scratch_types=[
            pltpu.VMEM((bpw,), jnp.int32),
            pltpu.VMEM((bpw, _E), jnp.float32),
            pltpu.VMEM((bpw,), jnp.int32),
            pltpu.VMEM((bpw, _E), jnp.float32),
            pltpu.SemaphoreType.DMA,
            pltpu.SemaphoreType.DMA,
        ],
        compiler_params=pltpu.CompilerParams(use_tc_tiling_on_sc=False),
    )
    def gather(uid_hbm, sid_hbm, ut_hbm, st_hbm, ue_hbm, se_hbm,
               uidx_v, urows_v, sidx_v, srows_v, usem, ssem):
        wid = lax.axis_index("s") * nc + lax.axis_index("c")
        base = wid * bpw
        pltpu.sync_copy(uid_hbm.at[pl.ds(base, bpw)], uidx_v)
        pltpu.sync_copy(sid_hbm.at[pl.ds(base, bpw)], sidx_v)
        ucps = []
        scps = []
        for j in range(nchunk):
            sl = pl.ds(j * _IDX_CHUNK, _IDX_CHUNK)
            ucps.append(pltpu.async_copy(
                ut_hbm.at[uidx_v.at[sl]], urows_v.at[sl], usem))
            scps.append(pltpu.async_copy(
                st_hbm.at[sidx_v.at[sl]], srows_v.at[sl], ssem))
        for cp in ucps:
            cp.wait()
        pltpu.sync_copy(urows_v, ue_hbm.at[pl.ds(base, bpw)])
        for cp in scps:
            cp.wait()
        pltpu.sync_copy(srows_v, se_hbm.at[pl.ds(base, bpw)])

    return gather


_gather = _make_gather()


def _mlp_body(ue_ref, se_ref, w1u_ref, w1s_ref, b1_ref, w2_ref, b2_ref,
              w3_ref, b3_ref, wo_ref, bo_ref, out_ref):
    x = jnp.dot(ue_ref[...], w1u_ref[...], preferred_element_type=jnp.float32)
    x = x + jnp.dot(se_ref[...], w1s_ref[...],
                    preferred_element_type=jnp.float32)
    h = jnp.maximum(x + b1_ref[...], 0.0)
    h = jnp.maximum(
        jnp.dot(h, w2_ref[...], preferred_element_type=jnp.float32)
        + b2_ref[...], 0.0)
    h = jnp.maximum(
        jnp.dot(h, w3_ref[...], preferred_element_type=jnp.float32)
        + b3_ref[...], 0.0)
    o = jnp.sum(h * wo_ref[...], axis=1, keepdims=True) + bo_ref[...]
    out_ref[...] = 1.0 / (1.0 + jnp.exp(-o))


def _mlp(ue, se, w1u, w1s, b1, w2t, b2, w3t, b3, wo_row, bo):
    bn = 2048
    grid = (_B // bn,)
    full = lambda shape: pl.BlockSpec(shape, lambda i: (0, 0))
    return pl.pallas_call(
        _mlp_body,
        grid=grid,
        in_specs=[
            pl.BlockSpec((bn, _E), lambda i: (i, 0)),
            pl.BlockSpec((bn, _E), lambda i: (i, 0)),
            full((_E, 128)),
            full((_E, 128)),
            full((1, 128)),
            full((128, 64)),
            full((1, 64)),
            full((64, 32)),
            full((1, 32)),
            full((1, 32)),
            full((1, 1)),
        ],
        out_specs=pl.BlockSpec((bn, 1), lambda i: (i, 0)),
        out_shape=jax.ShapeDtypeStruct((_B, 1), jnp.float32),
    )(ue, se, w1u, w1s, b1, w2t, b2, w3t, b3, wo_row, bo)


def kernel(user_ids, symbol_ids, user_table, symbol_table,
           W1, b1, W2, b2, W3, b3, Wo, bo):
    uids = user_ids.astype(jnp.int32)
    sids = symbol_ids.astype(jnp.int32)
    ue, se = _gather(uids, sids, user_table, symbol_table)
    w1t = W1.T
    return _mlp(ue, se, w1t[:_E], w1t[_E:], b1.reshape(1, -1),
                W2.T, b2.reshape(1, -1), W3.T, b3.reshape(1, -1),
                Wo.reshape(1, -1), bo.reshape(1, 1))


# speedup vs baseline: 2.0188x; 2.0188x over previous
import functools

import jax
import jax.numpy as jnp
from jax import lax
from jax.experimental import pallas as pl
from jax.experimental.pallas import tpu as pltpu
from jax.experimental.pallas import tpu_sc as plsc

_B = 16384
_E = 64
_C = 32


def _make_gather():
    info = plsc.get_sparse_core_info()
    nc, ns = info.num_cores, info.num_subcores
    nw = nc * ns
    bpw = _B // nw
    nch = bpw // _C
    npair = nch // 2

    mesh = plsc.VectorSubcoreMesh(core_axis_name="c", subcore_axis_name="s")

    @functools.partial(
        pl.kernel,
        out_type=(
            jax.ShapeDtypeStruct((_B, _E), jnp.float32),
            jax.ShapeDtypeStruct((_B, _E), jnp.float32),
        ),
        mesh=mesh,
        scratch_types=[
            pltpu.VMEM((bpw,), jnp.int32),
            pltpu.VMEM((bpw,), jnp.int32),
            pltpu.SMEM((bpw,), jnp.int32),
            pltpu.VMEM((_C, 8, _E), jnp.float32),
            pltpu.VMEM((_C, 8, _E), jnp.float32),
            pltpu.VMEM((_C, _E), jnp.float32),
            pltpu.VMEM((_C, _E), jnp.float32),
            pltpu.SemaphoreType.DMA,
            pltpu.SemaphoreType.DMA,
            pltpu.SemaphoreType.DMA,
            pltpu.SemaphoreType.DMA,
        ],
    )
    def gather(uid_hbm, sid_hbm, ut3, st3, ue_hbm, se_hbm,
               ids_v, tidx_v, ids_s, gb0, gb1, ob0, ob1,
               gsem0, gsem1, osem0, osem1):
        wid = lax.axis_index("s") * nc + lax.axis_index("c")
        base = wid * bpw
        gbufs = (gb0, gb1)
        obufs = (ob0, ob1)
        gsems = (gsem0, gsem1)
        osems = (osem0, osem1)

        def run_table(id_hbm, tab3, out_hbm):
            pltpu.sync_copy(id_hbm.at[pl.ds(base, bpw)], ids_v)

            def chunk_ids(ch):
                ids = []
                for v in range(_C // 16):
                    vec = ids_v[pl.ds(ch * _C + v * 16, 16)]
                    ids.extend(vec[k] for k in range(16))
                return ids

            def start_chunk(ch, b):
                ids = chunk_ids(ch)
                for k in range(_C):
                    tid = lax.shift_right_logical(ids[k], 3)
                    pltpu.async_copy(tab3.at[tid], gbufs[b].at[k], gsems[b])

            for b in range(2):
                start_chunk(b, b)

            def body(g, carry):
                for b in range(2):
                    ch = g * 2 + b
                    pltpu.make_async_copy(
                        tab3.at[pl.ds(0, _C)], gbufs[b], gsems[b]).wait()

                    @pl.when(g > 0)
                    def _():
                        pltpu.make_async_copy(
                            out_hbm.at[pl.ds(0, _C)], obufs[b],
                            osems[b]).wait()

                    ids = chunk_ids(ch)
                    for k in range(_C):
                        r = ids[k] & 7
                        for j in range(_E // 16):
                            jl = pl.ds(j * 16, 16)
                            obufs[b][k, jl] = gbufs[b][k, r, jl]
                    pltpu.async_copy(
                        obufs[b], out_hbm.at[pl.ds(base + ch * _C, _C)],
                        osems[b])

                    @pl.when(g < npair - 1)
                    def _():
                        start_chunk(ch + 2, b)
                return carry

            lax.fori_loop(0, npair, body, 0)
            for b in range(2):
                pltpu.make_async_copy(
                    out_hbm.at[pl.ds(0, _C)], obufs[b], osems[b]).wait()

        run_table(uid_hbm, ut3, ue_hbm)
        run_table(sid_hbm, st3, se_hbm)

    return gather


_gather = _make_gather()


def _mlp_body(ue_ref, se_ref, w1u_ref, w1s_ref, b1_ref, w2_ref, b2_ref,
              w3_ref, b3_ref, wo_ref, bo_ref, out_ref):
    x = jnp.dot(ue_ref[...], w1u_ref[...], preferred_element_type=jnp.float32)
    x = x + jnp.dot(se_ref[...], w1s_ref[...],
                    preferred_element_type=jnp.float32)
    h = jnp.maximum(x + b1_ref[...], 0.0)
    h = jnp.maximum(
        jnp.dot(h, w2_ref[...], preferred_element_type=jnp.float32)
        + b2_ref[...], 0.0)
    h = jnp.maximum(
        jnp.dot(h, w3_ref[...], preferred_element_type=jnp.float32)
        + b3_ref[...], 0.0)
    o = jnp.sum(h * wo_ref[...], axis=1, keepdims=True) + bo_ref[...]
    out_ref[...] = 1.0 / (1.0 + jnp.exp(-o))


def _mlp(ue, se, w1u, w1s, b1, w2t, b2, w3t, b3, wo_row, bo):
    bn = 2048
    grid = (_B // bn,)
    full = lambda shape: pl.BlockSpec(shape, lambda i: (0, 0))
    return pl.pallas_call(
        _mlp_body,
        grid=grid,
        in_specs=[
            pl.BlockSpec((bn, _E), lambda i: (i, 0)),
            pl.BlockSpec((bn, _E), lambda i: (i, 0)),
            full((_E, 128)),
            full((_E, 128)),
            full((1, 128)),
            full((128, 64)),
            full((1, 64)),
            full((64, 32)),
            full((1, 32)),
            full((1, 32)),
            full((1, 1)),
        ],
        out_specs=pl.BlockSpec((bn, 1), lambda i: (i, 0)),
        out_shape=jax.ShapeDtypeStruct((_B, 1), jnp.float32),
    )(ue, se, w1u, w1s, b1, w2t, b2, w3t, b3, wo_row, bo)


def kernel(user_ids, symbol_ids, user_table, symbol_table,
           W1, b1, W2, b2, W3, b3, Wo, bo):
    uids = user_ids.astype(jnp.int32)
    sids = symbol_ids.astype(jnp.int32)
    ut3 = user_table.reshape(-1, 8, _E)
    st3 = symbol_table.reshape(-1, 8, _E)
    ue, se = _gather(uids, sids, ut3, st3)
    w1t = W1.T
    return _mlp(ue, se, w1t[:_E], w1t[_E:], b1.reshape(1, -1),
                W2.T, b2.reshape(1, -1), W3.T, b3.reshape(1, -1),
                Wo.reshape(1, -1), bo.reshape(1, 1))
